# baseline (speedup 1.0000x reference)
import functools

import jax
import jax.numpy as jnp
from jax import lax
from jax.experimental import pallas as pl
from jax.experimental.pallas import tpu as pltpu
from jax.experimental.pallas import tpu_sc as plsc

FRAME_RATE = 12.5
F = 1500
N_ROWS = 32 * 448
NW = 32
RW = N_ROWS // NW
G = 16
NGROUPS = RW // G
NFULL = 93
RUN = 3
MAGIC = 0x5F3759DF


def _rsqrt_newton(x):
    i = plsc.bitcast(x, jnp.int32)
    y = plsc.bitcast(MAGIC - lax.shift_right_logical(i, 1), jnp.float32)
    for _ in range(3):
        y = y * (1.5 - 0.5 * x * y * y)
    return y


def _sumsq_ramp(n):
    return n * (n + 1.0) * (2.0 * n + 1.0) * (1.0 / 6.0)


def _sc_body(pred_hbm, ts_hbm, mask_hbm, num_hbm, den_hbm,
             tsbuf, maskbuf, buf0, buf1, numbuf, denbuf,
             sem0, sem1):
    c = lax.axis_index("c")
    s = lax.axis_index("s")
    wid = s * 2 + c
    row0 = wid * RW

    lane = lax.broadcasted_iota(jnp.int32, (16,), 0)
    lane_f = lane.astype(jnp.float32)

    pltpu.sync_copy(ts_hbm.at[pl.ds(row0 * 2, RW * 2)], tsbuf)
    pltpu.sync_copy(mask_hbm.at[pl.ds(row0, RW)], maskbuf)

    def dma_start(g, buf, sem):
        return pltpu.async_copy(
            pred_hbm.at[pl.ds(row0 + g * G, G), :], buf, sem)

    def dma_wait(buf, sem):
        pltpu.make_async_copy(
            pred_hbm.at[pl.ds(0, G), :], buf, sem).wait()

    dma_start(0, buf0, sem0)
    dma_start(1, buf1, sem1)

    def process_group(g, buf, num_acc, den_acc):
        gbase = g * G
        tidx = 2 * gbase + 2 * lane
        sv = plsc.load_gather(tsbuf, [tidx])
        ev = plsc.load_gather(tsbuf, [tidx + 1])
        sf = (sv * FRAME_RATE).astype(jnp.int32).astype(jnp.float32)
        sf = jnp.minimum(jnp.maximum(sf, 0.0), float(F - 1))
        ef = (ev * FRAME_RATE).astype(jnp.int32).astype(jnp.float32)
        ef = jnp.maximum(sf + 1.0, jnp.minimum(ef + 1.0, float(F)))

        n1 = jnp.minimum(jnp.maximum(4.0 - sf, 0.0), 4.0)
        n2 = jnp.minimum(jnp.maximum(ef - (F - 4.0), 0.0), 4.0)
        wsq = 25.0 * (ef - sf) + 60.0 - _sumsq_ramp(n1) - _sumsq_ramp(n2)

        av = 5.0 - sf
        bv = ef + 4.0
        sfi = sf.astype(jnp.int32)
        efi = ef.astype(jnp.int32)
        va_v = jnp.maximum(lax.shift_right_arithmetic(sfi - 4, 4), 0)
        vd_v = jnp.minimum(lax.shift_right_arithmetic(efi + 3, 4) + 1, NFULL)

        zero = jnp.zeros((16,), jnp.float32)
        dotv = zero
        psqv = zero
        for r in range(G):
            def qbody(j, accs, _r=r):
                off = j * 64
                new = []
                for u in range(4):
                    p = buf[_r, pl.ds(off + u * 16, 16)]
                    new.append(accs[u] + p * p)
                return tuple(new)

            q0, q1, q2, q3 = lax.fori_loop(
                0, 23, qbody, (zero, zero, zero, zero))
            p92 = buf[r, pl.ds(92 * 16, 16)]
            q0 = q0 + p92 * p92

            va = va_v[r]
            vd = vd_v[r]
            nq = lax.shift_right_arithmetic(vd - va + 3, 2)
            base = va * 16

            def dbody(j, carry, _r=r):
                rise, fall, dots = carry
                off = base + j * 64
                new = []
                for u in range(4):
                    ok = (off + u * 16) < NFULL * 16
                    p = buf[_r, pl.ds(jnp.minimum(off + u * 16,
                                                  (NFULL - 1) * 16), 16)]
                    p = jnp.where(ok, p, 0.0)
                    w = jnp.minimum(
                        jnp.minimum(rise + float(16 * u),
                                    fall - float(16 * u)), 5.0)
                    w = jnp.maximum(w, 0.0)
                    new.append(dots[u] + w * p)
                return rise + 64.0, fall - 64.0, tuple(new)

            rise0 = lane_f + av[r] + base.astype(jnp.float32)
            fall0 = bv[r] - base.astype(jnp.float32) - lane_f
            _, _, (d0, d1, d2, d3) = lax.fori_loop(
                0, nq, dbody, (rise0, fall0, (zero, zero, zero, zero)))

            p = buf[r, pl.ds(F - 16, 16)]
            p = jnp.where(lane >= 4, p, 0.0)
            rt = lane_f + av[r] + float(F - 16)
            ft = bv[r] - float(F - 16) - lane_f
            w = jnp.minimum(jnp.minimum(rt, ft), 5.0)
            w = jnp.maximum(w, 0.0)
            d0 = d0 + w * p
            q0 = q0 + p * p
            dot_r = jnp.sum((d0 + d1) + (d2 + d3))
            psq_r = jnp.sum((q0 + q1) + (q2 + q3))
            dotv = jnp.where(lane == r, dot_r, dotv)
            psqv = jnp.where(lane == r, psq_r, psqv)

        inv_pn = _rsqrt_newton(jnp.maximum(psqv, 1e-16))
        inv_gn = _rsqrt_newton(0.04 * wsq)
        cos = (0.2 * dotv) * inv_pn * inv_gn

        mv = maskbuf[pl.ds(gbase, 16)]
        return num_acc + (1.0 - cos) * mv, den_acc + mv

    def outer(k, carry):
        num_acc, den_acc = carry
        g0 = 2 * k
        dma_wait(buf0, sem0)
        num_acc, den_acc = process_group(g0, buf0, num_acc, den_acc)

        @pl.when(g0 + 2 < NGROUPS)
        def _():
            dma_start(g0 + 2, buf0, sem0)

        dma_wait(buf1, sem1)
        num_acc, den_acc = process_group(g0 + 1, buf1, num_acc, den_acc)

        @pl.when(g0 + 3 < NGROUPS)
        def _():
            dma_start(g0 + 3, buf1, sem1)

        return num_acc, den_acc

    zero = jnp.zeros((16,), jnp.float32)
    num_acc, den_acc = lax.fori_loop(0, NGROUPS // 2, outer, (zero, zero))

    numbuf[...] = num_acc
    denbuf[...] = den_acc
    pltpu.sync_copy(numbuf, num_hbm.at[pl.ds(wid * 16, 16)])
    pltpu.sync_copy(denbuf, den_hbm.at[pl.ds(wid * 16, 16)])


def _final_body(num_ref, den_ref, out_ref):
    num = jnp.sum(num_ref[...])
    den = jnp.sum(den_ref[...])
    out_ref[0, 0] = num / jnp.maximum(den, 1.0)


def kernel(predicted_attn, token_timestamps, attention_mask):
    B, T, Fdim = predicted_attn.shape
    pred = predicted_attn.reshape(B * T, Fdim)
    ts = token_timestamps.reshape(B * T * 2)
    mask = attention_mask.astype(jnp.float32).reshape(B * T)

    mesh = plsc.VectorSubcoreMesh(core_axis_name="c", subcore_axis_name="s")
    sc = functools.partial(
        pl.kernel,
        mesh=mesh,
        compiler_params=pltpu.CompilerParams(needs_layout_passes=False),
        out_type=(
            jax.ShapeDtypeStruct((NW * 16,), jnp.float32),
            jax.ShapeDtypeStruct((NW * 16,), jnp.float32),
        ),
        scratch_types=[
            pltpu.VMEM((RW * 2,), jnp.float32),
            pltpu.VMEM((RW,), jnp.float32),
            pltpu.VMEM((G, F), jnp.float32),
            pltpu.VMEM((G, F), jnp.float32),
            pltpu.VMEM((16,), jnp.float32),
            pltpu.VMEM((16,), jnp.float32),
            pltpu.SemaphoreType.DMA,
            pltpu.SemaphoreType.DMA,
        ],
    )(_sc_body)
    num, den = sc(pred, ts, mask)

    out = pl.pallas_call(
        _final_body,
        grid=(1,),
        in_specs=[
            pl.BlockSpec((4, 128), lambda i: (0, 0)),
            pl.BlockSpec((4, 128), lambda i: (0, 0)),
        ],
        out_specs=pl.BlockSpec(memory_space=pltpu.SMEM),
        out_shape=jax.ShapeDtypeStruct((1, 1), jnp.float32),
    )(num.reshape(4, 128), den.reshape(4, 128))
    return out[0, 0]


# speedup vs baseline: 1.2186x; 1.2186x over previous
import functools

import jax
import jax.numpy as jnp
from jax import lax
from jax.experimental import pallas as pl
from jax.experimental.pallas import tpu as pltpu
from jax.experimental.pallas import tpu_sc as plsc

FRAME_RATE = 12.5
F = 1500
N_ROWS = 32 * 448
NW = 32
RW = N_ROWS // NW
G = 16
NGROUPS = RW // G
NFULL = 93
RUN = 3
MAGIC = 0x5F3759DF


def _rsqrt_newton(x):
    i = plsc.bitcast(x, jnp.int32)
    y = plsc.bitcast(MAGIC - lax.shift_right_logical(i, 1), jnp.float32)
    for _ in range(3):
        y = y * (1.5 - 0.5 * x * y * y)
    return y


def _sumsq_ramp(n):
    return n * (n + 1.0) * (2.0 * n + 1.0) * (1.0 / 6.0)


def _sc_body(pred_hbm, ts_hbm, mask_hbm, num_hbm, den_hbm,
             tsbuf, maskbuf, buf0, buf1, numbuf, denbuf,
             sem0, sem1):
    c = lax.axis_index("c")
    s = lax.axis_index("s")
    wid = s * 2 + c
    row0 = wid * RW

    lane = lax.broadcasted_iota(jnp.int32, (16,), 0)
    lane_f = lane.astype(jnp.float32)

    pltpu.sync_copy(ts_hbm.at[pl.ds(row0 * 2, RW * 2)], tsbuf)
    pltpu.sync_copy(mask_hbm.at[pl.ds(row0, RW)], maskbuf)

    def dma_start(g, buf, sem):
        return pltpu.async_copy(
            pred_hbm.at[pl.ds(row0 + g * G, G), :], buf, sem)

    def dma_wait(buf, sem):
        pltpu.make_async_copy(
            pred_hbm.at[pl.ds(0, G), :], buf, sem).wait()

    dma_start(0, buf0, sem0)
    dma_start(1, buf1, sem1)

    def process_group(g, buf, num_acc, den_acc):
        gbase = g * G
        tidx = 2 * gbase + 2 * lane
        sv = plsc.load_gather(tsbuf, [tidx])
        ev = plsc.load_gather(tsbuf, [tidx + 1])
        sf = (sv * FRAME_RATE).astype(jnp.int32).astype(jnp.float32)
        sf = jnp.minimum(jnp.maximum(sf, 0.0), float(F - 1))
        ef = (ev * FRAME_RATE).astype(jnp.int32).astype(jnp.float32)
        ef = jnp.maximum(sf + 1.0, jnp.minimum(ef + 1.0, float(F)))

        n1 = jnp.minimum(jnp.maximum(4.0 - sf, 0.0), 4.0)
        n2 = jnp.minimum(jnp.maximum(ef - (F - 4.0), 0.0), 4.0)
        wsq = 25.0 * (ef - sf) + 60.0 - _sumsq_ramp(n1) - _sumsq_ramp(n2)

        av = 5.0 - sf
        bv = ef + 4.0

        zero = jnp.zeros((16,), jnp.float32)
        dotv = zero
        psqv = zero
        for r in range(G):
            rise0 = lane_f + av[r]
            fall0 = bv[r] - lane_f

            def fbody(j, carry, _r=r):
                rise, fall, accs = carry
                off = j * 128
                new = list(accs)
                for u in range(8):
                    p = buf[_r, pl.ds(off + u * 16, 16)]
                    w = jnp.minimum(
                        jnp.minimum(rise + float(16 * u),
                                    fall - float(16 * u)), 5.0)
                    w = jnp.maximum(w, 0.0)
                    d, q = new[u % 4]
                    new[u % 4] = (d + w * p, q + p * p)
                return (rise + 128.0, fall - 128.0, tuple(new))

            accs0 = ((zero, zero),) * 4
            rise_t, fall_t, accs = lax.fori_loop(
                0, 11, fbody, (rise0, fall0, accs0))
            accs = list(accs)

            for u in range(5):
                p = buf[r, pl.ds(88 * 16 + u * 16, 16)]
                w = jnp.minimum(
                    jnp.minimum(rise_t + float(16 * u),
                                fall_t - float(16 * u)), 5.0)
                w = jnp.maximum(w, 0.0)
                d, q = accs[u % 4]
                accs[u % 4] = (d + w * p, q + p * p)

            p = buf[r, pl.ds(F - 16, 16)]
            p = jnp.where(lane >= 4, p, 0.0)
            w = jnp.minimum(jnp.minimum(rise_t + 76.0, fall_t - 76.0), 5.0)
            w = jnp.maximum(w, 0.0)
            (d0, q0), (d1, q1), (d2, q2), (d3, q3) = accs
            d0 = d0 + w * p
            q0 = q0 + p * p
            dot_r = jnp.sum((d0 + d1) + (d2 + d3))
            psq_r = jnp.sum((q0 + q1) + (q2 + q3))
            dotv = jnp.where(lane == r, dot_r, dotv)
            psqv = jnp.where(lane == r, psq_r, psqv)

        inv_pn = _rsqrt_newton(jnp.maximum(psqv, 1e-16))
        inv_gn = _rsqrt_newton(0.04 * wsq)
        cos = (0.2 * dotv) * inv_pn * inv_gn

        mv = maskbuf[pl.ds(gbase, 16)]
        return num_acc + (1.0 - cos) * mv, den_acc + mv

    def outer(k, carry):
        num_acc, den_acc = carry
        g0 = 2 * k
        dma_wait(buf0, sem0)
        num_acc, den_acc = process_group(g0, buf0, num_acc, den_acc)

        @pl.when(g0 + 2 < NGROUPS)
        def _():
            dma_start(g0 + 2, buf0, sem0)

        dma_wait(buf1, sem1)
        num_acc, den_acc = process_group(g0 + 1, buf1, num_acc, den_acc)

        @pl.when(g0 + 3 < NGROUPS)
        def _():
            dma_start(g0 + 3, buf1, sem1)

        return num_acc, den_acc

    zero = jnp.zeros((16,), jnp.float32)
    num_acc, den_acc = lax.fori_loop(0, NGROUPS // 2, outer, (zero, zero))

    numbuf[...] = num_acc
    denbuf[...] = den_acc
    r_out = lax.shift_right_logical(wid, 3)
    c_out = (wid & 7) * 16
    pltpu.sync_copy(numbuf, num_hbm.at[r_out, pl.ds(c_out, 16)])
    pltpu.sync_copy(denbuf, den_hbm.at[r_out, pl.ds(c_out, 16)])


def _final_body(num_ref, den_ref, out_ref):
    num = jnp.sum(num_ref[...])
    den = jnp.sum(den_ref[...])
    out_ref[0, 0] = num / jnp.maximum(den, 1.0)


def kernel(predicted_attn, token_timestamps, attention_mask):
    B, T, Fdim = predicted_attn.shape
    pred = predicted_attn.reshape(B * T, Fdim)
    ts = token_timestamps.reshape(B * T * 2)
    mask = attention_mask.astype(jnp.float32).reshape(B * T)

    mesh = plsc.VectorSubcoreMesh(core_axis_name="c", subcore_axis_name="s")
    sc = functools.partial(
        pl.kernel,
        mesh=mesh,
        compiler_params=pltpu.CompilerParams(needs_layout_passes=False),
        out_type=(
            jax.ShapeDtypeStruct((4, 128), jnp.float32),
            jax.ShapeDtypeStruct((4, 128), jnp.float32),
        ),
        scratch_types=[
            pltpu.VMEM((RW * 2,), jnp.float32),
            pltpu.VMEM((RW,), jnp.float32),
            pltpu.VMEM((G, F), jnp.float32),
            pltpu.VMEM((G, F), jnp.float32),
            pltpu.VMEM((16,), jnp.float32),
            pltpu.VMEM((16,), jnp.float32),
            pltpu.SemaphoreType.DMA,
            pltpu.SemaphoreType.DMA,
        ],
    )(_sc_body)
    num, den = sc(pred, ts, mask)

    out = pl.pallas_call(
        _final_body,
        grid=(1,),
        in_specs=[
            pl.BlockSpec((4, 128), lambda i: (0, 0)),
            pl.BlockSpec((4, 128), lambda i: (0, 0)),
        ],
        out_specs=pl.BlockSpec(memory_space=pltpu.SMEM),
        out_shape=jax.ShapeDtypeStruct((1, 1), jnp.float32),
    )(num, den)
    return out[0, 0]


# speedup vs baseline: 1.3368x; 1.0970x over previous
import functools

import jax
import jax.numpy as jnp
from jax import lax
from jax.experimental import pallas as pl
from jax.experimental.pallas import tpu as pltpu
from jax.experimental.pallas import tpu_sc as plsc

FRAME_RATE = 12.5
F = 1500
N_ROWS = 32 * 448
NW = 32
RW = N_ROWS // NW
G = 16
NGROUPS = RW // G
NFULL = 93
RUN = 3
MAGIC = 0x5F3759DF


def _rsqrt_newton(x):
    i = plsc.bitcast(x, jnp.int32)
    y = plsc.bitcast(MAGIC - lax.shift_right_logical(i, 1), jnp.float32)
    for _ in range(3):
        y = y * (1.5 - 0.5 * x * y * y)
    return y


def _sumsq_ramp(n):
    return n * (n + 1.0) * (2.0 * n + 1.0) * (1.0 / 6.0)


def _sc_body(pred_hbm, ts_hbm, mask_hbm, num_hbm, den_hbm,
             tsbuf, maskbuf, buf0, buf1, numbuf, denbuf,
             sem0, sem1):
    c = lax.axis_index("c")
    s = lax.axis_index("s")
    wid = s * 2 + c
    row0 = wid * RW

    lane = lax.broadcasted_iota(jnp.int32, (16,), 0)
    lane_f = lane.astype(jnp.float32)

    pltpu.sync_copy(ts_hbm.at[pl.ds(row0 * 2, RW * 2)], tsbuf)
    pltpu.sync_copy(mask_hbm.at[pl.ds(row0, RW)], maskbuf)

    def dma_start(g, buf, sem):
        return pltpu.async_copy(
            pred_hbm.at[pl.ds(row0 + g * G, G), :], buf, sem)

    def dma_wait(buf, sem):
        pltpu.make_async_copy(
            pred_hbm.at[pl.ds(0, G), :], buf, sem).wait()

    dma_start(0, buf0, sem0)
    dma_start(1, buf1, sem1)

    def process_group(g, buf, num_acc, den_acc):
        gbase = g * G
        tidx = 2 * gbase + 2 * lane
        sv = plsc.load_gather(tsbuf, [tidx])
        ev = plsc.load_gather(tsbuf, [tidx + 1])
        sf = (sv * FRAME_RATE).astype(jnp.int32).astype(jnp.float32)
        sf = jnp.minimum(jnp.maximum(sf, 0.0), float(F - 1))
        ef = (ev * FRAME_RATE).astype(jnp.int32).astype(jnp.float32)
        ef = jnp.maximum(sf + 1.0, jnp.minimum(ef + 1.0, float(F)))

        n1 = jnp.minimum(jnp.maximum(4.0 - sf, 0.0), 4.0)
        n2 = jnp.minimum(jnp.maximum(ef - (F - 4.0), 0.0), 4.0)
        wsq = 25.0 * (ef - sf) + 60.0 - _sumsq_ramp(n1) - _sumsq_ramp(n2)

        av = 5.0 - sf
        bv = ef + 4.0

        zero = jnp.zeros((16,), jnp.float32)
        dotv = zero
        psqv = zero
        for r in range(G):
            rise0 = lane_f + av[r]
            fall0 = bv[r] - lane_f

            def fbody(j, carry, _r=r):
                rise, fall, accs = carry
                off = j * (16 * RUN)
                new = []
                for u in range(RUN):
                    p = buf[_r, pl.ds(off + u * 16, 16)]
                    w = jnp.minimum(
                        jnp.minimum(rise + float(16 * u),
                                    fall - float(16 * u)), 5.0)
                    w = jnp.maximum(w, 0.0)
                    d, q = accs[u]
                    new.append((d + w * p, q + p * p))
                return (rise + float(16 * RUN), fall - float(16 * RUN),
                        tuple(new))

            accs0 = ((zero, zero),) * RUN
            rise_t, fall_t, accs = lax.fori_loop(
                0, NFULL // RUN, fbody, (rise0, fall0, accs0))

            p = buf[r, pl.ds(F - 16, 16)]
            p = jnp.where(lane >= 4, p, 0.0)
            w = jnp.minimum(jnp.minimum(rise_t - 4.0, fall_t + 4.0), 5.0)
            w = jnp.maximum(w, 0.0)
            (d0, q0), (d1, q1), (d2, q2) = accs
            d0 = d0 + w * p
            q0 = q0 + p * p
            dot_r = jnp.sum((d0 + d1) + d2)
            psq_r = jnp.sum((q0 + q1) + q2)
            dotv = jnp.where(lane == r, dot_r, dotv)
            psqv = jnp.where(lane == r, psq_r, psqv)

        inv_pn = _rsqrt_newton(jnp.maximum(psqv, 1e-16))
        inv_gn = _rsqrt_newton(0.04 * wsq)
        cos = (0.2 * dotv) * inv_pn * inv_gn

        mv = maskbuf[pl.ds(gbase, 16)]
        return num_acc + (1.0 - cos) * mv, den_acc + mv

    def outer(k, carry):
        num_acc, den_acc = carry
        g0 = 2 * k
        dma_wait(buf0, sem0)
        num_acc, den_acc = process_group(g0, buf0, num_acc, den_acc)

        @pl.when(g0 + 2 < NGROUPS)
        def _():
            dma_start(g0 + 2, buf0, sem0)

        dma_wait(buf1, sem1)
        num_acc, den_acc = process_group(g0 + 1, buf1, num_acc, den_acc)

        @pl.when(g0 + 3 < NGROUPS)
        def _():
            dma_start(g0 + 3, buf1, sem1)

        return num_acc, den_acc

    zero = jnp.zeros((16,), jnp.float32)
    num_acc, den_acc = lax.fori_loop(0, NGROUPS // 2, outer, (zero, zero))

    numbuf[...] = num_acc
    denbuf[...] = den_acc
    r_out = lax.shift_right_logical(wid, 3)
    c_out = (wid & 7) * 16
    pltpu.sync_copy(numbuf, num_hbm.at[r_out, pl.ds(c_out, 16)])
    pltpu.sync_copy(denbuf, den_hbm.at[r_out, pl.ds(c_out, 16)])


def _final_body(num_ref, den_ref, out_ref):
    num = jnp.sum(num_ref[...])
    den = jnp.sum(den_ref[...])
    out_ref[0, 0] = num / jnp.maximum(den, 1.0)


def kernel(predicted_attn, token_timestamps, attention_mask):
    B, T, Fdim = predicted_attn.shape
    pred = predicted_attn.reshape(B * T, Fdim)
    ts = token_timestamps.reshape(B * T * 2)
    mask = attention_mask.astype(jnp.float32).reshape(B * T)

    mesh = plsc.VectorSubcoreMesh(core_axis_name="c", subcore_axis_name="s")
    sc = functools.partial(
        pl.kernel,
        mesh=mesh,
        compiler_params=pltpu.CompilerParams(needs_layout_passes=False),
        out_type=(
            jax.ShapeDtypeStruct((4, 128), jnp.float32),
            jax.ShapeDtypeStruct((4, 128), jnp.float32),
        ),
        scratch_types=[
            pltpu.VMEM((RW * 2,), jnp.float32),
            pltpu.VMEM((RW,), jnp.float32),
            pltpu.VMEM((G, F), jnp.float32),
            pltpu.VMEM((G, F), jnp.float32),
            pltpu.VMEM((16,), jnp.float32),
            pltpu.VMEM((16,), jnp.float32),
            pltpu.SemaphoreType.DMA,
            pltpu.SemaphoreType.DMA,
        ],
    )(_sc_body)
    num, den = sc(pred, ts, mask)

    out = pl.pallas_call(
        _final_body,
        grid=(1,),
        in_specs=[
            pl.BlockSpec((4, 128), lambda i: (0, 0)),
            pl.BlockSpec((4, 128), lambda i: (0, 0)),
        ],
        out_specs=pl.BlockSpec(memory_space=pltpu.SMEM),
        out_shape=jax.ShapeDtypeStruct((1, 1), jnp.float32),
    )(num, den)
    return out[0, 0]


# speedup vs baseline: 1.7297x; 1.2939x over previous
import functools

import jax
import jax.numpy as jnp
from jax import lax
from jax.experimental import pallas as pl
from jax.experimental.pallas import tpu as pltpu
from jax.experimental.pallas import tpu_sc as plsc

FRAME_RATE = 12.5
F = 1500
B_ALL = 32
T_LEN = 448
N_ROWS = B_ALL * T_LEN
SPLIT_B = 16
N_TC = SPLIT_B * T_LEN
NW = 32
RW = (N_ROWS - N_TC) // NW
G = 16
NGROUPS = RW // G
NFULL = 93
RUN = 3
MAGIC = 0x5F3759DF


def _rsqrt_newton(x):
    i = plsc.bitcast(x, jnp.int32)
    y = plsc.bitcast(MAGIC - lax.shift_right_logical(i, 1), jnp.float32)
    for _ in range(3):
        y = y * (1.5 - 0.5 * x * y * y)
    return y


def _sumsq_ramp(n):
    return n * (n + 1.0) * (2.0 * n + 1.0) * (1.0 / 6.0)


def _sc_body(pred_hbm, ts_hbm, mask_hbm, num_hbm, den_hbm,
             tsbuf, maskbuf, buf0, buf1, numbuf, denbuf,
             sem0, sem1):
    c = lax.axis_index("c")
    s = lax.axis_index("s")
    wid = s * 2 + c
    row0 = N_TC + wid * RW

    lane = lax.broadcasted_iota(jnp.int32, (16,), 0)
    lane_f = lane.astype(jnp.float32)

    pltpu.sync_copy(ts_hbm.at[pl.ds(row0 * 2, RW * 2)], tsbuf)
    pltpu.sync_copy(mask_hbm.at[pl.ds(row0, RW)], maskbuf)

    def dma_start(g, buf, sem):
        return pltpu.async_copy(
            pred_hbm.at[pl.ds(row0 + g * G, G), :], buf, sem)

    def dma_wait(buf, sem):
        pltpu.make_async_copy(
            pred_hbm.at[pl.ds(0, G), :], buf, sem).wait()

    dma_start(0, buf0, sem0)
    dma_start(1, buf1, sem1)

    def process_group(g, buf, num_acc, den_acc):
        gbase = g * G
        tidx = 2 * gbase + 2 * lane
        sv = plsc.load_gather(tsbuf, [tidx])
        ev = plsc.load_gather(tsbuf, [tidx + 1])
        sf = (sv * FRAME_RATE).astype(jnp.int32).astype(jnp.float32)
        sf = jnp.minimum(jnp.maximum(sf, 0.0), float(F - 1))
        ef = (ev * FRAME_RATE).astype(jnp.int32).astype(jnp.float32)
        ef = jnp.maximum(sf + 1.0, jnp.minimum(ef + 1.0, float(F)))

        n1 = jnp.minimum(jnp.maximum(4.0 - sf, 0.0), 4.0)
        n2 = jnp.minimum(jnp.maximum(ef - (F - 4.0), 0.0), 4.0)
        wsq = 25.0 * (ef - sf) + 60.0 - _sumsq_ramp(n1) - _sumsq_ramp(n2)

        av = 5.0 - sf
        bv = ef + 4.0

        zero = jnp.zeros((16,), jnp.float32)
        dotv = zero
        psqv = zero
        for r in range(G):
            rise0 = lane_f + av[r]
            fall0 = bv[r] - lane_f

            def fbody(j, carry, _r=r):
                rise, fall, accs = carry
                off = j * (16 * RUN)
                new = []
                for u in range(RUN):
                    p = buf[_r, pl.ds(off + u * 16, 16)]
                    w = jnp.minimum(
                        jnp.minimum(rise + float(16 * u),
                                    fall - float(16 * u)), 5.0)
                    w = jnp.maximum(w, 0.0)
                    d, q = accs[u]
                    new.append((d + w * p, q + p * p))
                return (rise + float(16 * RUN), fall - float(16 * RUN),
                        tuple(new))

            accs0 = ((zero, zero),) * RUN
            rise_t, fall_t, accs = lax.fori_loop(
                0, NFULL // RUN, fbody, (rise0, fall0, accs0))

            p = buf[r, pl.ds(F - 16, 16)]
            p = jnp.where(lane >= 4, p, 0.0)
            w = jnp.minimum(jnp.minimum(rise_t - 4.0, fall_t + 4.0), 5.0)
            w = jnp.maximum(w, 0.0)
            (d0, q0), (d1, q1), (d2, q2) = accs
            d0 = d0 + w * p
            q0 = q0 + p * p
            dot_r = jnp.sum((d0 + d1) + d2)
            psq_r = jnp.sum((q0 + q1) + q2)
            dotv = jnp.where(lane == r, dot_r, dotv)
            psqv = jnp.where(lane == r, psq_r, psqv)

        inv_pn = _rsqrt_newton(jnp.maximum(psqv, 1e-16))
        inv_gn = _rsqrt_newton(0.04 * wsq)
        cos = (0.2 * dotv) * inv_pn * inv_gn

        mv = maskbuf[pl.ds(gbase, 16)]
        return num_acc + (1.0 - cos) * mv, den_acc + mv

    def outer(k, carry):
        num_acc, den_acc = carry
        g0 = 2 * k
        dma_wait(buf0, sem0)
        num_acc, den_acc = process_group(g0, buf0, num_acc, den_acc)

        @pl.when(g0 + 2 < NGROUPS)
        def _():
            dma_start(g0 + 2, buf0, sem0)

        dma_wait(buf1, sem1)
        num_acc, den_acc = process_group(g0 + 1, buf1, num_acc, den_acc)

        @pl.when(g0 + 3 < NGROUPS)
        def _():
            dma_start(g0 + 3, buf1, sem1)

        return num_acc, den_acc

    zero = jnp.zeros((16,), jnp.float32)
    num_acc, den_acc = lax.fori_loop(0, NGROUPS // 2, outer, (zero, zero))

    numbuf[...] = num_acc
    denbuf[...] = den_acc
    r_out = lax.shift_right_logical(wid, 3)
    c_out = (wid & 7) * 16
    pltpu.sync_copy(numbuf, num_hbm.at[r_out, pl.ds(c_out, 16)])
    pltpu.sync_copy(denbuf, den_hbm.at[r_out, pl.ds(c_out, 16)])


def _tc_body(pred_ref, ts_ref, mask_ref, out_ref, acc_ref):
    i = pl.program_id(0)
    nb = pl.num_programs(0)

    @pl.when(i == 0)
    def _init():
        acc_ref[0] = 0.0
        acc_ref[1] = 0.0

    pred = pred_ref[...]
    Tt, Fdim = pred.shape
    ts = ts_ref[0]
    start = ts[:, 0:1]
    end = ts[:, 1:2]

    sf = jnp.clip(jnp.floor(start * FRAME_RATE), 0.0, float(Fdim - 1))
    ef0 = jnp.floor(end * FRAME_RATE)
    ef = jnp.maximum(sf + 1.0, jnp.minimum(ef0 + 1.0, float(Fdim)))

    frames = lax.broadcasted_iota(jnp.int32, (Tt, Fdim), 1).astype(
        jnp.float32)
    w = jnp.minimum(frames - (sf - 5.0), (ef + 4.0) - frames)
    w = jnp.clip(w, 0.0, 5.0)

    dot = jnp.sum(pred * w, axis=-1) * 0.2
    psq = jnp.sum(pred * pred, axis=-1)
    gsq = jnp.sum(w * w, axis=-1) * 0.04

    pn = jnp.maximum(jnp.sqrt(psq), 1e-8)
    gn = jnp.maximum(jnp.sqrt(gsq), 1e-8)
    cos = dot / (pn * gn)

    m = mask_ref[0, 0]
    acc_ref[0] += jnp.sum((1.0 - cos) * m)
    acc_ref[1] += jnp.sum(m)

    @pl.when(i == nb - 1)
    def _fin():
        out_ref[0] = acc_ref[0]
        out_ref[1] = acc_ref[1]


def _final_body(num_ref, den_ref, tc_ref, out_ref):
    num = jnp.sum(num_ref[...]) + tc_ref[0]
    den = jnp.sum(den_ref[...]) + tc_ref[1]
    out_ref[0, 0] = num / jnp.maximum(den, 1.0)


def kernel(predicted_attn, token_timestamps, attention_mask):
    B, T, Fdim = predicted_attn.shape
    pred2 = predicted_attn.reshape(B * T, Fdim)
    ts_flat = token_timestamps.reshape(B * T * 2)
    mask_flat = attention_mask.astype(jnp.float32).reshape(B * T)

    mesh = plsc.VectorSubcoreMesh(core_axis_name="c", subcore_axis_name="s")
    sc = functools.partial(
        pl.kernel,
        mesh=mesh,
        compiler_params=pltpu.CompilerParams(needs_layout_passes=False),
        out_type=(
            jax.ShapeDtypeStruct((4, 128), jnp.float32),
            jax.ShapeDtypeStruct((4, 128), jnp.float32),
        ),
        scratch_types=[
            pltpu.VMEM((RW * 2,), jnp.float32),
            pltpu.VMEM((RW,), jnp.float32),
            pltpu.VMEM((G, F), jnp.float32),
            pltpu.VMEM((G, F), jnp.float32),
            pltpu.VMEM((16,), jnp.float32),
            pltpu.VMEM((16,), jnp.float32),
            pltpu.SemaphoreType.DMA,
            pltpu.SemaphoreType.DMA,
        ],
    )(_sc_body)
    num, den = sc(pred2, ts_flat, mask_flat)

    Tt = 128
    NB = N_TC // Tt
    tc_part = pl.pallas_call(
        _tc_body,
        grid=(NB,),
        in_specs=[
            pl.BlockSpec((Tt, Fdim), lambda i: (i, 0)),
            pl.BlockSpec((1, Tt, 2), lambda i: (i, 0, 0)),
            pl.BlockSpec((1, 1, Tt), lambda i: (i, 0, 0)),
        ],
        out_specs=pl.BlockSpec(memory_space=pltpu.SMEM),
        out_shape=jax.ShapeDtypeStruct((2,), jnp.float32),
        scratch_shapes=[pltpu.SMEM((2,), jnp.float32)],
        compiler_params=pltpu.CompilerParams(
            dimension_semantics=("arbitrary",),
        ),
    )(
        pred2,
        ts_flat.reshape(B * T // Tt, Tt, 2),
        mask_flat.reshape(B * T // Tt, 1, Tt),
    )

    out = pl.pallas_call(
        _final_body,
        grid=(1,),
        in_specs=[
            pl.BlockSpec((4, 128), lambda i: (0, 0)),
            pl.BlockSpec((4, 128), lambda i: (0, 0)),
            pl.BlockSpec(memory_space=pltpu.SMEM),
        ],
        out_specs=pl.BlockSpec(memory_space=pltpu.SMEM),
        out_shape=jax.ShapeDtypeStruct((1, 1), jnp.float32),
    )(num, den, tc_part)
    return out[0, 0]


# speedup vs baseline: 1.7620x; 1.0187x over previous
import functools

import jax
import jax.numpy as jnp
from jax import lax
from jax.experimental import pallas as pl
from jax.experimental.pallas import tpu as pltpu
from jax.experimental.pallas import tpu_sc as plsc

FRAME_RATE = 12.5
F = 1500
B_ALL = 32
T_LEN = 448
N_ROWS = B_ALL * T_LEN
SPLIT_B = 16
N_TC = SPLIT_B * T_LEN
NW = 32
RW = (N_ROWS - N_TC) // NW
G = 16
NGROUPS = RW // G
NFULL = 93
RUN = 3
MAGIC = 0x5F3759DF


def _rsqrt_newton(x):
    i = plsc.bitcast(x, jnp.int32)
    y = plsc.bitcast(MAGIC - lax.shift_right_logical(i, 1), jnp.float32)
    for _ in range(3):
        y = y * (1.5 - 0.5 * x * y * y)
    return y


def _sumsq_ramp(n):
    return n * (n + 1.0) * (2.0 * n + 1.0) * (1.0 / 6.0)


def _sc_body(pred_hbm, ts_hbm, mask_hbm, num_hbm, den_hbm,
             tsbuf, maskbuf, buf0, buf1, numbuf, denbuf,
             sem0, sem1):
    c = lax.axis_index("c")
    s = lax.axis_index("s")
    wid = s * 2 + c
    row0 = N_TC + wid * RW

    lane = lax.broadcasted_iota(jnp.int32, (16,), 0)
    lane_f = lane.astype(jnp.float32)

    pltpu.sync_copy(ts_hbm.at[pl.ds(row0 * 2, RW * 2)], tsbuf)
    pltpu.sync_copy(mask_hbm.at[pl.ds(row0, RW)], maskbuf)

    def dma_start(g, buf, sem):
        return pltpu.async_copy(
            pred_hbm.at[pl.ds(row0 + g * G, G), :], buf, sem)

    def dma_wait(buf, sem):
        pltpu.make_async_copy(
            pred_hbm.at[pl.ds(0, G), :], buf, sem).wait()

    dma_start(0, buf0, sem0)
    dma_start(1, buf1, sem1)

    def process_group(g, buf, num_acc, den_acc):
        gbase = g * G
        tidx = 2 * gbase + 2 * lane
        sv = plsc.load_gather(tsbuf, [tidx])
        ev = plsc.load_gather(tsbuf, [tidx + 1])
        sf = (sv * FRAME_RATE).astype(jnp.int32).astype(jnp.float32)
        sf = jnp.minimum(jnp.maximum(sf, 0.0), float(F - 1))
        ef = (ev * FRAME_RATE).astype(jnp.int32).astype(jnp.float32)
        ef = jnp.maximum(sf + 1.0, jnp.minimum(ef + 1.0, float(F)))

        n1 = jnp.minimum(jnp.maximum(4.0 - sf, 0.0), 4.0)
        n2 = jnp.minimum(jnp.maximum(ef - (F - 4.0), 0.0), 4.0)
        wsq = 25.0 * (ef - sf) + 60.0 - _sumsq_ramp(n1) - _sumsq_ramp(n2)

        av = 5.0 - sf
        bv = ef + 4.0

        zero = jnp.zeros((16,), jnp.float32)
        dotv = zero
        psqv = zero
        for r in range(G):
            rise0 = lane_f + av[r]
            fall0 = bv[r] - lane_f

            def fbody(j, carry, _r=r):
                rise, fall, accs = carry
                off = j * (16 * RUN)
                new = []
                for u in range(RUN):
                    p = buf[_r, pl.ds(off + u * 16, 16)]
                    w = jnp.minimum(
                        jnp.minimum(rise + float(16 * u),
                                    fall - float(16 * u)), 5.0)
                    w = jnp.maximum(w, 0.0)
                    d, q = accs[u]
                    new.append((d + w * p, q + p * p))
                return (rise + float(16 * RUN), fall - float(16 * RUN),
                        tuple(new))

            accs0 = ((zero, zero),) * RUN
            rise_t, fall_t, accs = lax.fori_loop(
                0, NFULL // RUN, fbody, (rise0, fall0, accs0))

            p = buf[r, pl.ds(F - 16, 16)]
            p = jnp.where(lane >= 4, p, 0.0)
            w = jnp.minimum(jnp.minimum(rise_t - 4.0, fall_t + 4.0), 5.0)
            w = jnp.maximum(w, 0.0)
            (d0, q0), (d1, q1), (d2, q2) = accs
            d0 = d0 + w * p
            q0 = q0 + p * p
            dot_r = jnp.sum((d0 + d1) + d2)
            psq_r = jnp.sum((q0 + q1) + q2)
            dotv = jnp.where(lane == r, dot_r, dotv)
            psqv = jnp.where(lane == r, psq_r, psqv)

        inv_pn = _rsqrt_newton(jnp.maximum(psqv, 1e-16))
        inv_gn = _rsqrt_newton(0.04 * wsq)
        cos = (0.2 * dotv) * inv_pn * inv_gn

        mv = maskbuf[pl.ds(gbase, 16)]
        return num_acc + (1.0 - cos) * mv, den_acc + mv

    def outer(k, carry):
        num_acc, den_acc = carry
        g0 = 2 * k
        dma_wait(buf0, sem0)
        num_acc, den_acc = process_group(g0, buf0, num_acc, den_acc)

        @pl.when(g0 + 2 < NGROUPS)
        def _():
            dma_start(g0 + 2, buf0, sem0)

        dma_wait(buf1, sem1)
        num_acc, den_acc = process_group(g0 + 1, buf1, num_acc, den_acc)

        @pl.when(g0 + 3 < NGROUPS)
        def _():
            dma_start(g0 + 3, buf1, sem1)

        return num_acc, den_acc

    zero = jnp.zeros((16,), jnp.float32)
    num_acc, den_acc = lax.fori_loop(0, NGROUPS // 2, outer, (zero, zero))

    numbuf[...] = num_acc
    denbuf[...] = den_acc
    r_out = lax.shift_right_logical(wid, 3)
    c_out = (wid & 7) * 16
    pltpu.sync_copy(numbuf, num_hbm.at[r_out, pl.ds(c_out, 16)])
    pltpu.sync_copy(denbuf, den_hbm.at[r_out, pl.ds(c_out, 16)])


def _tc_body(pred_ref, ts_ref, mask_ref, out_ref, acc_ref):
    i = pl.program_id(0)
    nb = pl.num_programs(0)

    @pl.when(i == 0)
    def _init():
        acc_ref[0] = 0.0
        acc_ref[1] = 0.0

    pred = pred_ref[...]
    Tt, Fdim = pred.shape
    ts = ts_ref[0]
    start = ts[:, 0:1]
    end = ts[:, 1:2]

    sf = jnp.clip(jnp.floor(start * FRAME_RATE), 0.0, float(Fdim - 1))
    ef0 = jnp.floor(end * FRAME_RATE)
    ef = jnp.maximum(sf + 1.0, jnp.minimum(ef0 + 1.0, float(Fdim)))

    frames = lax.broadcasted_iota(jnp.int32, (Tt, Fdim), 1).astype(
        jnp.float32)
    w = jnp.minimum(frames - (sf - 5.0), (ef + 4.0) - frames)
    w = jnp.clip(w, 0.0, 5.0)

    dot = jnp.sum(pred * w, axis=-1) * 0.2
    psq = jnp.sum(pred * pred, axis=-1)
    n1 = jnp.clip(4.0 - sf, 0.0, 4.0)
    n2 = jnp.clip(ef - (float(Fdim) - 4.0), 0.0, 4.0)
    gsq = ((25.0 * (ef - sf) + 60.0
            - _sumsq_ramp(n1) - _sumsq_ramp(n2)) * 0.04)[:, 0]

    pn = jnp.maximum(jnp.sqrt(psq), 1e-8)
    gn = jnp.maximum(jnp.sqrt(gsq), 1e-8)
    cos = dot / (pn * gn)

    m = mask_ref[0, 0]
    acc_ref[0] += jnp.sum((1.0 - cos) * m)
    acc_ref[1] += jnp.sum(m)

    @pl.when(i == nb - 1)
    def _fin():
        out_ref[0] = acc_ref[0]
        out_ref[1] = acc_ref[1]


def _final_body(num_ref, den_ref, tc_ref, out_ref):
    num = jnp.sum(num_ref[...]) + tc_ref[0]
    den = jnp.sum(den_ref[...]) + tc_ref[1]
    out_ref[0, 0] = num / jnp.maximum(den, 1.0)


def kernel(predicted_attn, token_timestamps, attention_mask):
    B, T, Fdim = predicted_attn.shape
    pred2 = predicted_attn.reshape(B * T, Fdim)
    ts_flat = token_timestamps.reshape(B * T * 2)
    mask_flat = attention_mask.astype(jnp.float32).reshape(B * T)

    mesh = plsc.VectorSubcoreMesh(core_axis_name="c", subcore_axis_name="s")
    sc = functools.partial(
        pl.kernel,
        mesh=mesh,
        compiler_params=pltpu.CompilerParams(needs_layout_passes=False),
        out_type=(
            jax.ShapeDtypeStruct((4, 128), jnp.float32),
            jax.ShapeDtypeStruct((4, 128), jnp.float32),
        ),
        scratch_types=[
            pltpu.VMEM((RW * 2,), jnp.float32),
            pltpu.VMEM((RW,), jnp.float32),
            pltpu.VMEM((G, F), jnp.float32),
            pltpu.VMEM((G, F), jnp.float32),
            pltpu.VMEM((16,), jnp.float32),
            pltpu.VMEM((16,), jnp.float32),
            pltpu.SemaphoreType.DMA,
            pltpu.SemaphoreType.DMA,
        ],
    )(_sc_body)
    num, den = sc(pred2, ts_flat, mask_flat)

    Tt = 128
    NB = N_TC // Tt
    tc_part = pl.pallas_call(
        _tc_body,
        grid=(NB,),
        in_specs=[
            pl.BlockSpec((Tt, Fdim), lambda i: (i, 0)),
            pl.BlockSpec((1, Tt, 2), lambda i: (i, 0, 0)),
            pl.BlockSpec((1, 1, Tt), lambda i: (i, 0, 0)),
        ],
        out_specs=pl.BlockSpec(memory_space=pltpu.SMEM),
        out_shape=jax.ShapeDtypeStruct((2,), jnp.float32),
        scratch_shapes=[pltpu.SMEM((2,), jnp.float32)],
        compiler_params=pltpu.CompilerParams(
            dimension_semantics=("arbitrary",),
        ),
    )(
        pred2,
        ts_flat.reshape(B * T // Tt, Tt, 2),
        mask_flat.reshape(B * T // Tt, 1, Tt),
    )

    out = pl.pallas_call(
        _final_body,
        grid=(1,),
        in_specs=[
            pl.BlockSpec((4, 128), lambda i: (0, 0)),
            pl.BlockSpec((4, 128), lambda i: (0, 0)),
            pl.BlockSpec(memory_space=pltpu.SMEM),
        ],
        out_specs=pl.BlockSpec(memory_space=pltpu.SMEM),
        out_shape=jax.ShapeDtypeStruct((1, 1), jnp.float32),
    )(num, den, tc_part)
    return out[0, 0]


# speedup vs baseline: 1.8576x; 1.0543x over previous
import functools

import jax
import jax.numpy as jnp
from jax import lax
from jax.experimental import pallas as pl
from jax.experimental.pallas import tpu as pltpu
from jax.experimental.pallas import tpu_sc as plsc

FRAME_RATE = 12.5
F = 1500
B_ALL = 32
T_LEN = 448
N_ROWS = B_ALL * T_LEN
N_TC = 6144
NW = 32
RW = (N_ROWS - N_TC) // NW
G = 16
NGROUPS = RW // G
NFULL = 93
RUN = 3
MAGIC = 0x5F3759DF


def _rsqrt_newton(x):
    i = plsc.bitcast(x, jnp.int32)
    y = plsc.bitcast(MAGIC - lax.shift_right_logical(i, 1), jnp.float32)
    for _ in range(3):
        y = y * (1.5 - 0.5 * x * y * y)
    return y


def _sumsq_ramp(n):
    return n * (n + 1.0) * (2.0 * n + 1.0) * (1.0 / 6.0)


def _sc_body(pred_hbm, ts_hbm, mask_hbm, num_hbm, den_hbm,
             tsbuf, maskbuf, buf0, buf1, numbuf, denbuf,
             sem0, sem1):
    c = lax.axis_index("c")
    s = lax.axis_index("s")
    wid = s * 2 + c
    row0 = N_TC + wid * RW

    lane = lax.broadcasted_iota(jnp.int32, (16,), 0)
    lane_f = lane.astype(jnp.float32)

    pltpu.sync_copy(ts_hbm.at[pl.ds(row0 * 2, RW * 2)], tsbuf)
    pltpu.sync_copy(mask_hbm.at[pl.ds(row0, RW)], maskbuf)

    def dma_start(g, buf, sem):
        return pltpu.async_copy(
            pred_hbm.at[pl.ds(row0 + g * G, G), :], buf, sem)

    def dma_wait(buf, sem):
        pltpu.make_async_copy(
            pred_hbm.at[pl.ds(0, G), :], buf, sem).wait()

    dma_start(0, buf0, sem0)
    dma_start(1, buf1, sem1)

    def process_group(g, buf, num_acc, den_acc):
        gbase = g * G
        tidx = 2 * gbase + 2 * lane
        sv = plsc.load_gather(tsbuf, [tidx])
        ev = plsc.load_gather(tsbuf, [tidx + 1])
        sf = (sv * FRAME_RATE).astype(jnp.int32).astype(jnp.float32)
        sf = jnp.minimum(jnp.maximum(sf, 0.0), float(F - 1))
        ef = (ev * FRAME_RATE).astype(jnp.int32).astype(jnp.float32)
        ef = jnp.maximum(sf + 1.0, jnp.minimum(ef + 1.0, float(F)))

        n1 = jnp.minimum(jnp.maximum(4.0 - sf, 0.0), 4.0)
        n2 = jnp.minimum(jnp.maximum(ef - (F - 4.0), 0.0), 4.0)
        wsq = 25.0 * (ef - sf) + 60.0 - _sumsq_ramp(n1) - _sumsq_ramp(n2)

        av = 5.0 - sf
        bv = ef + 4.0

        zero = jnp.zeros((16,), jnp.float32)
        dotv = zero
        psqv = zero
        for r in range(G):
            rise0 = lane_f + av[r]
            fall0 = bv[r] - lane_f

            def fbody(j, carry, _r=r):
                rise, fall, accs = carry
                off = j * (16 * RUN)
                new = []
                for u in range(RUN):
                    p = buf[_r, pl.ds(off + u * 16, 16)]
                    w = jnp.minimum(
                        jnp.minimum(rise + float(16 * u),
                                    fall - float(16 * u)), 5.0)
                    w = jnp.maximum(w, 0.0)
                    d, q = accs[u]
                    new.append((d + w * p, q + p * p))
                return (rise + float(16 * RUN), fall - float(16 * RUN),
                        tuple(new))

            accs0 = ((zero, zero),) * RUN
            rise_t, fall_t, accs = lax.fori_loop(
                0, NFULL // RUN, fbody, (rise0, fall0, accs0))

            p = buf[r, pl.ds(F - 16, 16)]
            p = jnp.where(lane >= 4, p, 0.0)
            w = jnp.minimum(jnp.minimum(rise_t - 4.0, fall_t + 4.0), 5.0)
            w = jnp.maximum(w, 0.0)
            (d0, q0), (d1, q1), (d2, q2) = accs
            d0 = d0 + w * p
            q0 = q0 + p * p
            dot_r = jnp.sum((d0 + d1) + d2)
            psq_r = jnp.sum((q0 + q1) + q2)
            dotv = jnp.where(lane == r, dot_r, dotv)
            psqv = jnp.where(lane == r, psq_r, psqv)

        inv_pn = _rsqrt_newton(jnp.maximum(psqv, 1e-16))
        inv_gn = _rsqrt_newton(0.04 * wsq)
        cos = (0.2 * dotv) * inv_pn * inv_gn

        mv = maskbuf[pl.ds(gbase, 16)]
        return num_acc + (1.0 - cos) * mv, den_acc + mv

    def outer(k, carry):
        num_acc, den_acc = carry
        g0 = 2 * k
        dma_wait(buf0, sem0)
        num_acc, den_acc = process_group(g0, buf0, num_acc, den_acc)

        @pl.when(g0 + 2 < NGROUPS)
        def _():
            dma_start(g0 + 2, buf0, sem0)

        dma_wait(buf1, sem1)
        num_acc, den_acc = process_group(g0 + 1, buf1, num_acc, den_acc)

        @pl.when(g0 + 3 < NGROUPS)
        def _():
            dma_start(g0 + 3, buf1, sem1)

        return num_acc, den_acc

    zero = jnp.zeros((16,), jnp.float32)
    num_acc, den_acc = lax.fori_loop(0, NGROUPS // 2, outer, (zero, zero))

    numbuf[...] = num_acc
    denbuf[...] = den_acc
    r_out = lax.shift_right_logical(wid, 3)
    c_out = (wid & 7) * 16
    pltpu.sync_copy(numbuf, num_hbm.at[r_out, pl.ds(c_out, 16)])
    pltpu.sync_copy(denbuf, den_hbm.at[r_out, pl.ds(c_out, 16)])


def _tc_body(pred_ref, ts_ref, mask_ref, out_ref, acc_ref):
    i = pl.program_id(0)
    nb = pl.num_programs(0)

    @pl.when(i == 0)
    def _init():
        acc_ref[0] = 0.0
        acc_ref[1] = 0.0

    pred = pred_ref[...]
    Tt, Fdim = pred.shape
    ts = ts_ref[0]
    start = ts[:, 0:1]
    end = ts[:, 1:2]

    sf = jnp.clip(jnp.floor(start * FRAME_RATE), 0.0, float(Fdim - 1))
    ef0 = jnp.floor(end * FRAME_RATE)
    ef = jnp.maximum(sf + 1.0, jnp.minimum(ef0 + 1.0, float(Fdim)))

    frames = lax.broadcasted_iota(jnp.int32, (Tt, Fdim), 1).astype(
        jnp.float32)
    w = jnp.minimum(frames - (sf - 5.0), (ef + 4.0) - frames)
    w = jnp.clip(w, 0.0, 5.0)

    dot = jnp.sum(pred * w, axis=-1) * 0.2
    psq = jnp.sum(pred * pred, axis=-1)
    n1 = jnp.clip(4.0 - sf, 0.0, 4.0)
    n2 = jnp.clip(ef - (float(Fdim) - 4.0), 0.0, 4.0)
    gsq = ((25.0 * (ef - sf) + 60.0
            - _sumsq_ramp(n1) - _sumsq_ramp(n2)) * 0.04)[:, 0]

    pn = jnp.maximum(jnp.sqrt(psq), 1e-8)
    gn = jnp.maximum(jnp.sqrt(gsq), 1e-8)
    cos = dot / (pn * gn)

    m = mask_ref[0, 0]
    acc_ref[0] += jnp.sum((1.0 - cos) * m)
    acc_ref[1] += jnp.sum(m)

    @pl.when(i == nb - 1)
    def _fin():
        out_ref[0] = acc_ref[0]
        out_ref[1] = acc_ref[1]


def _final_body(num_ref, den_ref, tc_ref, out_ref):
    num = jnp.sum(num_ref[...]) + tc_ref[0]
    den = jnp.sum(den_ref[...]) + tc_ref[1]
    out_ref[0, 0] = num / jnp.maximum(den, 1.0)


def kernel(predicted_attn, token_timestamps, attention_mask):
    B, T, Fdim = predicted_attn.shape
    pred2 = predicted_attn.reshape(B * T, Fdim)
    ts_flat = token_timestamps.reshape(B * T * 2)
    mask_flat = attention_mask.astype(jnp.float32).reshape(B * T)

    mesh = plsc.VectorSubcoreMesh(core_axis_name="c", subcore_axis_name="s")
    sc = functools.partial(
        pl.kernel,
        mesh=mesh,
        compiler_params=pltpu.CompilerParams(needs_layout_passes=False),
        out_type=(
            jax.ShapeDtypeStruct((4, 128), jnp.float32),
            jax.ShapeDtypeStruct((4, 128), jnp.float32),
        ),
        scratch_types=[
            pltpu.VMEM((RW * 2,), jnp.float32),
            pltpu.VMEM((RW,), jnp.float32),
            pltpu.VMEM((G, F), jnp.float32),
            pltpu.VMEM((G, F), jnp.float32),
            pltpu.VMEM((16,), jnp.float32),
            pltpu.VMEM((16,), jnp.float32),
            pltpu.SemaphoreType.DMA,
            pltpu.SemaphoreType.DMA,
        ],
    )(_sc_body)
    num, den = sc(pred2, ts_flat, mask_flat)

    Tt = 128
    NB = N_TC // Tt
    tc_part = pl.pallas_call(
        _tc_body,
        grid=(NB,),
        in_specs=[
            pl.BlockSpec((Tt, Fdim), lambda i: (i, 0)),
            pl.BlockSpec((1, Tt, 2), lambda i: (i, 0, 0)),
            pl.BlockSpec((1, 1, Tt), lambda i: (i, 0, 0)),
        ],
        out_specs=pl.BlockSpec(memory_space=pltpu.SMEM),
        out_shape=jax.ShapeDtypeStruct((2,), jnp.float32),
        scratch_shapes=[pltpu.SMEM((2,), jnp.float32)],
        compiler_params=pltpu.CompilerParams(
            dimension_semantics=("arbitrary",),
        ),
    )(
        pred2,
        ts_flat.reshape(B * T // Tt, Tt, 2),
        mask_flat.reshape(B * T // Tt, 1, Tt),
    )

    out = pl.pallas_call(
        _final_body,
        grid=(1,),
        in_specs=[
            pl.BlockSpec((4, 128), lambda i: (0, 0)),
            pl.BlockSpec((4, 128), lambda i: (0, 0)),
            pl.BlockSpec(memory_space=pltpu.SMEM),
        ],
        out_specs=pl.BlockSpec(memory_space=pltpu.SMEM),
        out_shape=jax.ShapeDtypeStruct((1, 1), jnp.float32),
    )(num, den, tc_part)
    return out[0, 0]


# speedup vs baseline: 1.9603x; 1.0553x over previous
import functools

import jax
import jax.numpy as jnp
from jax import lax
from jax.experimental import pallas as pl
from jax.experimental.pallas import tpu as pltpu
from jax.experimental.pallas import tpu_sc as plsc

FRAME_RATE = 12.5
F = 1500
B_ALL = 32
T_LEN = 448
N_ROWS = B_ALL * T_LEN
N_TC = 7168
NW = 32
RW = (N_ROWS - N_TC) // NW
G = 16
NGROUPS = RW // G
NFULL = 93
RUN = 3
MAGIC = 0x5F3759DF


def _rsqrt_newton(x):
    i = plsc.bitcast(x, jnp.int32)
    y = plsc.bitcast(MAGIC - lax.shift_right_logical(i, 1), jnp.float32)
    for _ in range(3):
        y = y * (1.5 - 0.5 * x * y * y)
    return y


def _sumsq_ramp(n):
    return n * (n + 1.0) * (2.0 * n + 1.0) * (1.0 / 6.0)


def _sc_body(pred_hbm, ts_hbm, mask_hbm, num_hbm, den_hbm,
             tsbuf, maskbuf, buf0, buf1, numbuf, denbuf,
             sem0, sem1):
    c = lax.axis_index("c")
    s = lax.axis_index("s")
    wid = s * 2 + c
    row0 = N_TC + wid * RW

    lane = lax.broadcasted_iota(jnp.int32, (16,), 0)
    lane_f = lane.astype(jnp.float32)

    pltpu.sync_copy(ts_hbm.at[pl.ds(row0 * 2, RW * 2)], tsbuf)
    pltpu.sync_copy(mask_hbm.at[pl.ds(row0, RW)], maskbuf)

    def dma_start(g, buf, sem):
        return pltpu.async_copy(
            pred_hbm.at[pl.ds(row0 + g * G, G), :], buf, sem)

    def dma_wait(buf, sem):
        pltpu.make_async_copy(
            pred_hbm.at[pl.ds(0, G), :], buf, sem).wait()

    dma_start(0, buf0, sem0)
    dma_start(1, buf1, sem1)

    def process_group(g, buf, num_acc, den_acc):
        gbase = g * G
        tidx = 2 * gbase + 2 * lane
        sv = plsc.load_gather(tsbuf, [tidx])
        ev = plsc.load_gather(tsbuf, [tidx + 1])
        sf = (sv * FRAME_RATE).astype(jnp.int32).astype(jnp.float32)
        sf = jnp.minimum(jnp.maximum(sf, 0.0), float(F - 1))
        ef = (ev * FRAME_RATE).astype(jnp.int32).astype(jnp.float32)
        ef = jnp.maximum(sf + 1.0, jnp.minimum(ef + 1.0, float(F)))

        n1 = jnp.minimum(jnp.maximum(4.0 - sf, 0.0), 4.0)
        n2 = jnp.minimum(jnp.maximum(ef - (F - 4.0), 0.0), 4.0)
        wsq = 25.0 * (ef - sf) + 60.0 - _sumsq_ramp(n1) - _sumsq_ramp(n2)

        av = 5.0 - sf
        bv = ef + 4.0

        zero = jnp.zeros((16,), jnp.float32)
        dotv = zero
        psqv = zero
        for r in range(G):
            rise0 = lane_f + av[r]
            fall0 = bv[r] - lane_f

            def fbody(j, carry, _r=r):
                rise, fall, accs = carry
                off = j * (16 * RUN)
                new = []
                for u in range(RUN):
                    p = buf[_r, pl.ds(off + u * 16, 16)]
                    w = jnp.minimum(
                        jnp.minimum(rise + float(16 * u),
                                    fall - float(16 * u)), 5.0)
                    w = jnp.maximum(w, 0.0)
                    d, q = accs[u]
                    new.append((d + w * p, q + p * p))
                return (rise + float(16 * RUN), fall - float(16 * RUN),
                        tuple(new))

            accs0 = ((zero, zero),) * RUN
            rise_t, fall_t, accs = lax.fori_loop(
                0, NFULL // RUN, fbody, (rise0, fall0, accs0))

            p = buf[r, pl.ds(F - 16, 16)]
            p = jnp.where(lane >= 4, p, 0.0)
            w = jnp.minimum(jnp.minimum(rise_t - 4.0, fall_t + 4.0), 5.0)
            w = jnp.maximum(w, 0.0)
            (d0, q0), (d1, q1), (d2, q2) = accs
            d0 = d0 + w * p
            q0 = q0 + p * p
            dot_r = jnp.sum((d0 + d1) + d2)
            psq_r = jnp.sum((q0 + q1) + q2)
            dotv = jnp.where(lane == r, dot_r, dotv)
            psqv = jnp.where(lane == r, psq_r, psqv)

        inv_pn = _rsqrt_newton(jnp.maximum(psqv, 1e-16))
        inv_gn = _rsqrt_newton(0.04 * wsq)
        cos = (0.2 * dotv) * inv_pn * inv_gn

        mv = maskbuf[pl.ds(gbase, 16)]
        return num_acc + (1.0 - cos) * mv, den_acc + mv

    def outer(k, carry):
        num_acc, den_acc = carry
        g0 = 2 * k
        dma_wait(buf0, sem0)
        num_acc, den_acc = process_group(g0, buf0, num_acc, den_acc)

        @pl.when(g0 + 2 < NGROUPS)
        def _():
            dma_start(g0 + 2, buf0, sem0)

        dma_wait(buf1, sem1)
        num_acc, den_acc = process_group(g0 + 1, buf1, num_acc, den_acc)

        @pl.when(g0 + 3 < NGROUPS)
        def _():
            dma_start(g0 + 3, buf1, sem1)

        return num_acc, den_acc

    zero = jnp.zeros((16,), jnp.float32)
    num_acc, den_acc = lax.fori_loop(0, NGROUPS // 2, outer, (zero, zero))

    numbuf[...] = num_acc
    denbuf[...] = den_acc
    r_out = lax.shift_right_logical(wid, 3)
    c_out = (wid & 7) * 16
    pltpu.sync_copy(numbuf, num_hbm.at[r_out, pl.ds(c_out, 16)])
    pltpu.sync_copy(denbuf, den_hbm.at[r_out, pl.ds(c_out, 16)])


def _tc_body(pred_ref, ts_ref, mask_ref, out_ref, acc_ref):
    i = pl.program_id(0)
    nb = pl.num_programs(0)

    @pl.when(i == 0)
    def _init():
        acc_ref[0] = 0.0
        acc_ref[1] = 0.0

    pred = pred_ref[...]
    Tt, Fdim = pred.shape
    ts = ts_ref[0]
    start = ts[:, 0:1]
    end = ts[:, 1:2]

    sf = jnp.clip(jnp.floor(start * FRAME_RATE), 0.0, float(Fdim - 1))
    ef0 = jnp.floor(end * FRAME_RATE)
    ef = jnp.maximum(sf + 1.0, jnp.minimum(ef0 + 1.0, float(Fdim)))

    frames = lax.broadcasted_iota(jnp.int32, (Tt, Fdim), 1).astype(
        jnp.float32)
    w = jnp.minimum(frames - (sf - 5.0), (ef + 4.0) - frames)
    w = jnp.clip(w, 0.0, 5.0)

    dot = jnp.sum(pred * w, axis=-1) * 0.2
    psq = jnp.sum(pred * pred, axis=-1)
    n1 = jnp.clip(4.0 - sf, 0.0, 4.0)
    n2 = jnp.clip(ef - (float(Fdim) - 4.0), 0.0, 4.0)
    gsq = ((25.0 * (ef - sf) + 60.0
            - _sumsq_ramp(n1) - _sumsq_ramp(n2)) * 0.04)[:, 0]

    pn = jnp.maximum(jnp.sqrt(psq), 1e-8)
    gn = jnp.maximum(jnp.sqrt(gsq), 1e-8)
    cos = dot / (pn * gn)

    m = mask_ref[0, 0]
    acc_ref[0] += jnp.sum((1.0 - cos) * m)
    acc_ref[1] += jnp.sum(m)

    @pl.when(i == nb - 1)
    def _fin():
        out_ref[0] = acc_ref[0]
        out_ref[1] = acc_ref[1]


def _final_body(num_ref, den_ref, tc_ref, out_ref):
    num = jnp.sum(num_ref[...]) + tc_ref[0]
    den = jnp.sum(den_ref[...]) + tc_ref[1]
    out_ref[0, 0] = num / jnp.maximum(den, 1.0)


def kernel(predicted_attn, token_timestamps, attention_mask):
    B, T, Fdim = predicted_attn.shape
    pred2 = predicted_attn.reshape(B * T, Fdim)
    ts_flat = token_timestamps.reshape(B * T * 2)
    mask_flat = attention_mask.astype(jnp.float32).reshape(B * T)

    mesh = plsc.VectorSubcoreMesh(core_axis_name="c", subcore_axis_name="s")
    sc = functools.partial(
        pl.kernel,
        mesh=mesh,
        compiler_params=pltpu.CompilerParams(needs_layout_passes=False),
        out_type=(
            jax.ShapeDtypeStruct((4, 128), jnp.float32),
            jax.ShapeDtypeStruct((4, 128), jnp.float32),
        ),
        scratch_types=[
            pltpu.VMEM((RW * 2,), jnp.float32),
            pltpu.VMEM((RW,), jnp.float32),
            pltpu.VMEM((G, F), jnp.float32),
            pltpu.VMEM((G, F), jnp.float32),
            pltpu.VMEM((16,), jnp.float32),
            pltpu.VMEM((16,), jnp.float32),
            pltpu.SemaphoreType.DMA,
            pltpu.SemaphoreType.DMA,
        ],
    )(_sc_body)
    num, den = sc(pred2, ts_flat, mask_flat)

    Tt = T_LEN
    NB = N_TC // Tt
    tc_part = pl.pallas_call(
        _tc_body,
        grid=(NB,),
        in_specs=[
            pl.BlockSpec((Tt, Fdim), lambda i: (i, 0)),
            pl.BlockSpec((1, Tt, 2), lambda i: (i, 0, 0)),
            pl.BlockSpec((1, 1, Tt), lambda i: (i, 0, 0)),
        ],
        out_specs=pl.BlockSpec(memory_space=pltpu.SMEM),
        out_shape=jax.ShapeDtypeStruct((2,), jnp.float32),
        scratch_shapes=[pltpu.SMEM((2,), jnp.float32)],
        compiler_params=pltpu.CompilerParams(
            dimension_semantics=("arbitrary",),
        ),
    )(
        pred2,
        token_timestamps,
        mask_flat.reshape(B, 1, T),
    )

    out = pl.pallas_call(
        _final_body,
        grid=(1,),
        in_specs=[
            pl.BlockSpec((4, 128), lambda i: (0, 0)),
            pl.BlockSpec((4, 128), lambda i: (0, 0)),
            pl.BlockSpec(memory_space=pltpu.SMEM),
        ],
        out_specs=pl.BlockSpec(memory_space=pltpu.SMEM),
        out_shape=jax.ShapeDtypeStruct((1, 1), jnp.float32),
    )(num, den, tc_part)
    return out[0, 0]


# speedup vs baseline: 2.0977x; 1.0701x over previous
import functools

import jax
import jax.numpy as jnp
from jax import lax
from jax.experimental import pallas as pl
from jax.experimental.pallas import tpu as pltpu
from jax.experimental.pallas import tpu_sc as plsc

FRAME_RATE = 12.5
F = 1500
B_ALL = 32
T_LEN = 448
N_ROWS = B_ALL * T_LEN
N_TC = 7168
NW = 32
RW = (N_ROWS - N_TC) // NW
G = 16
NGROUPS = RW // G
NFULL = 93
RUN = 3
MAGIC = 0x5F3759DF


def _rsqrt_newton(x):
    i = plsc.bitcast(x, jnp.int32)
    y = plsc.bitcast(MAGIC - lax.shift_right_logical(i, 1), jnp.float32)
    for _ in range(3):
        y = y * (1.5 - 0.5 * x * y * y)
    return y


def _sumsq_ramp(n):
    return n * (n + 1.0) * (2.0 * n + 1.0) * (1.0 / 6.0)


def _sc_body(pred_hbm, ts_hbm, mask_hbm, num_hbm, den_hbm,
             tsbuf, maskbuf, buf0, buf1, numbuf, denbuf,
             sem0, sem1):
    c = lax.axis_index("c")
    s = lax.axis_index("s")
    wid = s * 2 + c
    row0 = N_TC + wid * RW
    bw = N_TC // T_LEN + lax.shift_right_logical(wid, 1)
    t0 = (wid & 1) * RW

    lane = lax.broadcasted_iota(jnp.int32, (16,), 0)
    lane_f = lane.astype(jnp.float32)

    pltpu.sync_copy(ts_hbm.at[bw, pl.ds(t0, RW), :], tsbuf)
    pltpu.sync_copy(mask_hbm.at[pl.ds(row0, RW)], maskbuf)

    def dma_start(g, buf, sem):
        return pltpu.async_copy(
            pred_hbm.at[pl.ds(row0 + g * G, G), :], buf, sem)

    def dma_wait(buf, sem):
        pltpu.make_async_copy(
            pred_hbm.at[pl.ds(0, G), :], buf, sem).wait()

    dma_start(0, buf0, sem0)
    dma_start(1, buf1, sem1)

    def process_group(g, buf, num_acc, den_acc):
        gbase = g * G
        ridx = gbase + lane
        zc = jnp.zeros((16,), jnp.int32)
        sv = plsc.load_gather(tsbuf, [ridx, zc])
        ev = plsc.load_gather(tsbuf, [ridx, zc + 1])
        sf = (sv * FRAME_RATE).astype(jnp.int32).astype(jnp.float32)
        sf = jnp.minimum(jnp.maximum(sf, 0.0), float(F - 1))
        ef = (ev * FRAME_RATE).astype(jnp.int32).astype(jnp.float32)
        ef = jnp.maximum(sf + 1.0, jnp.minimum(ef + 1.0, float(F)))

        n1 = jnp.minimum(jnp.maximum(4.0 - sf, 0.0), 4.0)
        n2 = jnp.minimum(jnp.maximum(ef - (F - 4.0), 0.0), 4.0)
        wsq = 25.0 * (ef - sf) + 60.0 - _sumsq_ramp(n1) - _sumsq_ramp(n2)

        av = 5.0 - sf
        bv = ef + 4.0

        zero = jnp.zeros((16,), jnp.float32)
        dotv = zero
        psqv = zero
        for r in range(G):
            rise0 = lane_f + av[r]
            fall0 = bv[r] - lane_f

            def fbody(j, carry, _r=r):
                rise, fall, accs = carry
                off = j * (16 * RUN)
                new = []
                for u in range(RUN):
                    p = buf[_r, pl.ds(off + u * 16, 16)]
                    w = jnp.minimum(
                        jnp.minimum(rise + float(16 * u),
                                    fall - float(16 * u)), 5.0)
                    w = jnp.maximum(w, 0.0)
                    d, q = accs[u]
                    new.append((d + w * p, q + p * p))
                return (rise + float(16 * RUN), fall - float(16 * RUN),
                        tuple(new))

            accs0 = ((zero, zero),) * RUN
            rise_t, fall_t, accs = lax.fori_loop(
                0, NFULL // RUN, fbody, (rise0, fall0, accs0))

            p = buf[r, pl.ds(F - 16, 16)]
            p = jnp.where(lane >= 4, p, 0.0)
            w = jnp.minimum(jnp.minimum(rise_t - 4.0, fall_t + 4.0), 5.0)
            w = jnp.maximum(w, 0.0)
            (d0, q0), (d1, q1), (d2, q2) = accs
            d0 = d0 + w * p
            q0 = q0 + p * p
            dot_r = jnp.sum((d0 + d1) + d2)
            psq_r = jnp.sum((q0 + q1) + q2)
            dotv = jnp.where(lane == r, dot_r, dotv)
            psqv = jnp.where(lane == r, psq_r, psqv)

        inv_pn = _rsqrt_newton(jnp.maximum(psqv, 1e-16))
        inv_gn = _rsqrt_newton(0.04 * wsq)
        cos = (0.2 * dotv) * inv_pn * inv_gn

        mv = maskbuf[pl.ds(gbase, 16)]
        return num_acc + (1.0 - cos) * mv, den_acc + mv

    def outer(k, carry):
        num_acc, den_acc = carry
        g0 = 2 * k
        dma_wait(buf0, sem0)
        num_acc, den_acc = process_group(g0, buf0, num_acc, den_acc)

        @pl.when(g0 + 2 < NGROUPS)
        def _():
            dma_start(g0 + 2, buf0, sem0)

        dma_wait(buf1, sem1)
        num_acc, den_acc = process_group(g0 + 1, buf1, num_acc, den_acc)

        @pl.when(g0 + 3 < NGROUPS)
        def _():
            dma_start(g0 + 3, buf1, sem1)

        return num_acc, den_acc

    zero = jnp.zeros((16,), jnp.float32)
    num_acc, den_acc = lax.fori_loop(0, NGROUPS // 2, outer, (zero, zero))

    numbuf[...] = num_acc
    denbuf[...] = den_acc
    r_out = lax.shift_right_logical(wid, 3)
    c_out = (wid & 7) * 16
    pltpu.sync_copy(numbuf, num_hbm.at[r_out, pl.ds(c_out, 16)])
    pltpu.sync_copy(denbuf, den_hbm.at[r_out, pl.ds(c_out, 16)])


def _tc_body(pred_ref, ts_ref, mask_ref, out_ref, acc_ref):
    i = pl.program_id(0)
    nb = pl.num_programs(0)

    @pl.when(i == 0)
    def _init():
        acc_ref[0] = 0.0
        acc_ref[1] = 0.0

    pred = pred_ref[...]
    Tt, Fdim = pred.shape
    ts = ts_ref[0]
    start = ts[:, 0:1]
    end = ts[:, 1:2]

    sf = jnp.clip(jnp.floor(start * FRAME_RATE), 0.0, float(Fdim - 1))
    ef0 = jnp.floor(end * FRAME_RATE)
    ef = jnp.maximum(sf + 1.0, jnp.minimum(ef0 + 1.0, float(Fdim)))

    frames = lax.broadcasted_iota(jnp.int32, (Tt, Fdim), 1).astype(
        jnp.float32)
    w = jnp.minimum(frames - (sf - 5.0), (ef + 4.0) - frames)
    w = jnp.clip(w, 0.0, 5.0)

    dot = jnp.sum(pred * w, axis=-1) * 0.2
    psq = jnp.sum(pred * pred, axis=-1)
    n1 = jnp.clip(4.0 - sf, 0.0, 4.0)
    n2 = jnp.clip(ef - (float(Fdim) - 4.0), 0.0, 4.0)
    gsq = ((25.0 * (ef - sf) + 60.0
            - _sumsq_ramp(n1) - _sumsq_ramp(n2)) * 0.04)[:, 0]

    pn = jnp.maximum(jnp.sqrt(psq), 1e-8)
    gn = jnp.maximum(jnp.sqrt(gsq), 1e-8)
    cos = dot / (pn * gn)

    m = mask_ref[0, 0]
    acc_ref[0] += jnp.sum((1.0 - cos) * m)
    acc_ref[1] += jnp.sum(m)

    @pl.when(i == nb - 1)
    def _fin():
        out_ref[0] = acc_ref[0]
        out_ref[1] = acc_ref[1]


def _final_body(num_ref, den_ref, tc_ref, out_ref):
    num = jnp.sum(num_ref[...]) + tc_ref[0]
    den = jnp.sum(den_ref[...]) + tc_ref[1]
    out_ref[0, 0] = num / jnp.maximum(den, 1.0)


def kernel(predicted_attn, token_timestamps, attention_mask):
    B, T, Fdim = predicted_attn.shape
    pred2 = predicted_attn.reshape(B * T, Fdim)
    mask_flat = attention_mask.astype(jnp.float32).reshape(B * T)

    mesh = plsc.VectorSubcoreMesh(core_axis_name="c", subcore_axis_name="s")
    sc = functools.partial(
        pl.kernel,
        mesh=mesh,
        compiler_params=pltpu.CompilerParams(needs_layout_passes=False),
        out_type=(
            jax.ShapeDtypeStruct((4, 128), jnp.float32),
            jax.ShapeDtypeStruct((4, 128), jnp.float32),
        ),
        scratch_types=[
            pltpu.VMEM((RW, 2), jnp.float32),
            pltpu.VMEM((RW,), jnp.float32),
            pltpu.VMEM((G, F), jnp.float32),
            pltpu.VMEM((G, F), jnp.float32),
            pltpu.VMEM((16,), jnp.float32),
            pltpu.VMEM((16,), jnp.float32),
            pltpu.SemaphoreType.DMA,
            pltpu.SemaphoreType.DMA,
        ],
    )(_sc_body)
    num, den = sc(pred2, token_timestamps, mask_flat)

    Tt = T_LEN
    NB = N_TC // Tt
    tc_part = pl.pallas_call(
        _tc_body,
        grid=(NB,),
        in_specs=[
            pl.BlockSpec((Tt, Fdim), lambda i: (i, 0)),
            pl.BlockSpec((1, Tt, 2), lambda i: (i, 0, 0)),
            pl.BlockSpec((1, 1, Tt), lambda i: (i, 0, 0)),
        ],
        out_specs=pl.BlockSpec(memory_space=pltpu.SMEM),
        out_shape=jax.ShapeDtypeStruct((2,), jnp.float32),
        scratch_shapes=[pltpu.SMEM((2,), jnp.float32)],
        compiler_params=pltpu.CompilerParams(
            dimension_semantics=("arbitrary",),
        ),
    )(
        pred2,
        token_timestamps,
        mask_flat.reshape(B, 1, T),
    )

    out = pl.pallas_call(
        _final_body,
        grid=(1,),
        in_specs=[
            pl.BlockSpec((4, 128), lambda i: (0, 0)),
            pl.BlockSpec((4, 128), lambda i: (0, 0)),
            pl.BlockSpec(memory_space=pltpu.SMEM),
        ],
        out_specs=pl.BlockSpec(memory_space=pltpu.SMEM),
        out_shape=jax.ShapeDtypeStruct((1, 1), jnp.float32),
    )(num, den, tc_part)
    return out[0, 0]
